# baseline (speedup 1.0000x reference)
import jax
import jax.numpy as jnp
from jax import lax
from jax.experimental import pallas as pl
from jax.experimental.pallas import tpu as pltpu
from jax.experimental.pallas import tpu_sc as plsc

N = 10000
E = 320000
IN_CH = 256
HID = 256
OUT_CH = 128

NC = 2
NS = 16
K = 40
NROWS = E // K
NBUF = 8
NBUF_PROP = 8
CR_PROP = 25
NCH_PROP = NROWS // NS // CR_PROP
CR_DEG = 25
NCH_DEG = NROWS // (NC * NS) // CR_DEG
CP_TILES = 10
CP_ROWS = N // CP_TILES
DEG_W = 128

_mesh = plsc.VectorSubcoreMesh(
    core_axis_name="c", subcore_axis_name="s", num_cores=NC, num_subcores=NS
)


def _zero_acc(sid, zeros_hbm, acc_sh):
    @pl.when(sid < CP_TILES)
    def _():
        pltpu.sync_copy(zeros_hbm,
                        acc_sh.at[pl.ds(sid * CP_ROWS, CP_ROWS)])


def _copy_out(cid, sid, out_hbm, acc_sh):
    @pl.when(sid < CP_TILES)
    def _():
        pltpu.sync_copy(acc_sh.at[pl.ds(sid * CP_ROWS, CP_ROWS)],
                        out_hbm.at[pl.ds(cid * N + sid * CP_ROWS, CP_ROWS)])


EPT = E // (NC * NS)


def _deg_body(dst_hbm, deg_out, dstv, acc):
    cid = lax.axis_index("c")
    sid = lax.axis_index("s")
    tid = cid * NS + sid

    zeros16 = jnp.zeros((16,), jnp.float32)

    def z(i, c):
        acc[pl.ds(i * 16, 16)] = zeros16
        return c

    lax.fori_loop(0, N // 16, z, 0)
    pltpu.sync_copy(dst_hbm.at[pl.ds(tid * EPT, EPT)], dstv)

    ones16 = jnp.ones((16,), jnp.float32)

    def step(i, c):
        idx = dstv[pl.ds(i * 16, 16)]
        plsc.addupdate_scatter(acc, [idx], ones16)
        return c

    lax.fori_loop(0, EPT // 16, step, 0)
    pltpu.sync_copy(acc, deg_out.at[pl.ds(tid * N, N)])


_deg_call = pl.kernel(
    _deg_body,
    out_type=jax.ShapeDtypeStruct((NC * NS * N,), jnp.float32),
    mesh=_mesh,
    scratch_types=[
        pltpu.VMEM((EPT,), jnp.int32),
        pltpu.VMEM((N,), jnp.float32),
    ],
    compiler_params=pltpu.CompilerParams(needs_layout_passes=False),
)


def _run_waves(cr, g_hbm, acc_sh, src_v, dst_v, bufs, gsems, ssems):
    nb = len(bufs)
    nwave = cr // nb
    ntail = cr - nwave * nb

    def wave(q, c2):
        base = q * nb
        for b in range(nb):
            j = base + b
            pltpu.make_async_copy(g_hbm.at[src_v.at[j]], bufs[b],
                                  gsems[b]).wait()
            pltpu.async_copy(bufs[b], acc_sh.at[dst_v.at[j]], ssems[b],
                             add=True)
        for b in range(nb):
            jn = base + b + nb

            @pl.when(jn < cr)
            def _(b=b, jn=jn):
                pltpu.make_async_copy(bufs[b], acc_sh.at[dst_v.at[0]],
                                      ssems[b]).wait()
                pltpu.async_copy(g_hbm.at[src_v.at[jn]], bufs[b],
                                 gsems[b])

        return c2

    lax.fori_loop(0, nwave, wave, 0)
    for b in range(ntail):
        j = nwave * nb + b
        pltpu.make_async_copy(g_hbm.at[src_v.at[j]], bufs[b],
                              gsems[b]).wait()
        pltpu.async_copy(bufs[b], acc_sh.at[dst_v.at[j]], ssems[b],
                         add=True)


def _drain_scatters(acc_sh, dst_v, bufs, ssems):
    for b in range(len(bufs)):
        pltpu.make_async_copy(bufs[b], acc_sh.at[dst_v.at[0]],
                              ssems[b]).wait()


def _prop_pipelined(nch, cr, src_row, dst_row, g_hbm, src_hbm, dst_hbm,
                    acc_sh, src_v, dst_v, bufs, gsems, ssems, is0, is1):
    nb = len(bufs)
    pltpu.sync_copy(src_hbm.at[src_row(0)], src_v)
    pltpu.sync_copy(dst_hbm.at[dst_row(0)], dst_v)
    for b in range(nb):
        pltpu.async_copy(g_hbm.at[src_v.at[b]], bufs[b], gsems[b])
    plsc.subcore_barrier()
    _run_waves(cr, g_hbm, acc_sh, src_v, dst_v, bufs, gsems, ssems)

    def chunk(t, carry):
        pltpu.async_copy(src_hbm.at[src_row(t)], src_v, is0)
        _drain_scatters(acc_sh, dst_v, bufs, ssems)
        pltpu.async_copy(dst_hbm.at[dst_row(t)], dst_v, is1)
        pltpu.make_async_copy(src_hbm.at[src_row(t)], src_v, is0).wait()
        for b in range(nb):
            pltpu.async_copy(g_hbm.at[src_v.at[b]], bufs[b], gsems[b])
        pltpu.make_async_copy(dst_hbm.at[dst_row(t)], dst_v, is1).wait()
        _run_waves(cr, g_hbm, acc_sh, src_v, dst_v, bufs, gsems, ssems)
        return carry

    lax.fori_loop(1, nch, chunk, 0)
    _drain_scatters(acc_sh, dst_v, bufs, ssems)


def _make_prop_body(nbuf, edge_split):
    def body(g_hbm, src_hbm, dst_hbm, zeros_hbm, out_hbm, *scr):
        src_v, dst_v = scr[0], scr[1]
        bufs = list(scr[2:2 + nbuf])
        acc_sh = scr[2 + nbuf]
        is0, is1 = scr[3 + nbuf], scr[4 + nbuf]
        gsems = list(scr[5 + nbuf:5 + 2 * nbuf])
        ssems = list(scr[5 + 2 * nbuf:5 + 3 * nbuf])

        cid = lax.axis_index("c")
        sid = lax.axis_index("s")
        tid = cid * NS + sid

        _zero_acc(sid, zeros_hbm, acc_sh)

        if edge_split:
            nch, cr = NCH_DEG, CR_DEG

            def src_row(t):
                return tid * nch + t

            dst_row = src_row
        else:
            nch, cr = NCH_PROP, CR_PROP

            def src_row(t):
                return tid * nch + t

            def dst_row(t):
                return sid * nch + t

        _prop_pipelined(nch, cr, src_row, dst_row, g_hbm, src_hbm, dst_hbm,
                        acc_sh, src_v, dst_v, bufs, gsems, ssems, is0, is1)
        plsc.subcore_barrier()
        _copy_out(cid, sid, out_hbm, acc_sh)

    return body


def _prop_scratch(c_width, cr, nbuf):
    return (
        [pltpu.VMEM((cr, K), jnp.int32),
         pltpu.VMEM((cr, K), jnp.int32)]
        + [pltpu.VMEM((K, c_width), jnp.float32) for _ in range(nbuf)]
        + [pltpu.VMEM_SHARED((N, c_width), jnp.float32)]
        + [pltpu.SemaphoreType.DMA for _ in range(2 + 2 * nbuf)]
    )


_prop128 = pl.kernel(
    _make_prop_body(NBUF_PROP, edge_split=False),
    out_type=jax.ShapeDtypeStruct((NC * N, HID // 2), jnp.float32),
    mesh=_mesh,
    scratch_types=_prop_scratch(HID // 2, CR_PROP, NBUF_PROP),
)


_prop_es = pl.kernel(
    _make_prop_body(NBUF, edge_split=True),
    out_type=jax.ShapeDtypeStruct((NC * N, OUT_CH), jnp.float32),
    mesh=_mesh,
    scratch_types=_prop_scratch(OUT_CH, CR_DEG, NBUF),
)


_B = 1000
_NB = N // _B
_PREC = lax.Precision.DEFAULT


def _tc1_body(dp_ref, x_ref, w_ref, g_ref, d8_ref):
    deg = jnp.sum(dp_ref[0], axis=0)[:, None] + 1.0
    d = lax.rsqrt(deg)
    h = jnp.dot(x_ref[...], w_ref[...], preferred_element_type=jnp.float32,
                precision=_PREC)
    g_ref[0] = h[:, :HID // 2] * d
    g_ref[1] = h[:, HID // 2:] * d
    d8_ref[...] = jnp.broadcast_to(d, (_B, 8))


def _tc1(degp, x, w1):
    return pl.pallas_call(
        _tc1_body,
        grid=(_NB,),
        in_specs=[
            pl.BlockSpec((1, NC * NS, _B), lambda i: (i, 0, 0)),
            pl.BlockSpec((_B, IN_CH), lambda i: (i, 0)),
            pl.BlockSpec((IN_CH, HID), lambda i: (0, 0)),
        ],
        out_specs=[
            pl.BlockSpec((2, _B, HID // 2), lambda i: (0, i, 0)),
            pl.BlockSpec((_B, 8), lambda i: (i, 0)),
        ],
        out_shape=[
            jax.ShapeDtypeStruct((2, N, HID // 2), jnp.float32),
            jax.ShapeDtypeStruct((N, 8), jnp.float32),
        ],
    )(degp, x, w1)


def _tc2_body(scat_ref, g_ref, d8_ref, b1_ref, w2_ref, out_ref):
    d = d8_ref[:, 0:1]
    h1a = jnp.maximum(d * (scat_ref[0] + g_ref[0]) + b1_ref[0:1, :], 0.0)
    h1b = jnp.maximum(d * (scat_ref[1] + g_ref[1]) + b1_ref[1:2, :], 0.0)
    res = (jnp.dot(h1a, w2_ref[0:HID // 2, :],
                   preferred_element_type=jnp.float32, precision=_PREC)
           + jnp.dot(h1b, w2_ref[HID // 2:, :],
                     preferred_element_type=jnp.float32, precision=_PREC))
    out_ref[...] = res * d


def _tc2(scat1, g1, d8, b1r, w2):
    return pl.pallas_call(
        _tc2_body,
        grid=(_NB,),
        in_specs=[
            pl.BlockSpec((2, _B, HID // 2), lambda i: (0, i, 0)),
            pl.BlockSpec((2, _B, HID // 2), lambda i: (0, i, 0)),
            pl.BlockSpec((_B, 8), lambda i: (i, 0)),
            pl.BlockSpec((2, HID // 2), lambda i: (0, 0)),
            pl.BlockSpec((HID, OUT_CH), lambda i: (0, 0)),
        ],
        out_specs=pl.BlockSpec((_B, OUT_CH), lambda i: (i, 0)),
        out_shape=jax.ShapeDtypeStruct((N, OUT_CH), jnp.float32),
    )(scat1, g1, d8, b1r, w2)


def _tc3_body(scat_ref, g_ref, d8_ref, b2_ref, z_ref):
    d = d8_ref[:, 0:1]
    z_ref[...] = d * (scat_ref[0] + scat_ref[1] + g_ref[...]) + b2_ref[0:1, :]


def _tc3(scat2, g2, d8, b2r):
    return pl.pallas_call(
        _tc3_body,
        grid=(_NB,),
        in_specs=[
            pl.BlockSpec((2, _B, OUT_CH), lambda i: (0, i, 0)),
            pl.BlockSpec((_B, OUT_CH), lambda i: (i, 0)),
            pl.BlockSpec((_B, 8), lambda i: (i, 0)),
            pl.BlockSpec((1, OUT_CH), lambda i: (0, 0)),
        ],
        out_specs=pl.BlockSpec((_B, OUT_CH), lambda i: (i, 0)),
        out_shape=jax.ShapeDtypeStruct((N, OUT_CH), jnp.float32),
    )(scat2, g2, d8, b2r)


def kernel(x, edge_index, W1, b1, W2, b2):
    src = edge_index[0].astype(jnp.int32)
    dst = edge_index[1].astype(jnp.int32)
    src_off = jnp.concatenate([src, src + N]).reshape(
        NC * NS * NCH_PROP, CR_PROP, K)
    dst_prop = dst.reshape(NS * NCH_PROP, CR_PROP, K)
    src_es = src.reshape(NC * NS * NCH_DEG, CR_DEG, K)
    dst_es = dst.reshape(NC * NS * NCH_DEG, CR_DEG, K)

    zeros_128 = jnp.zeros((CP_ROWS, HID // 2), jnp.float32)

    degp = _deg_call(dst).reshape(NC * NS, _NB, _B).transpose(1, 0, 2)

    g1, d8 = _tc1(degp, x, W1)
    scat1 = _prop128(g1.reshape(NC * N, HID // 2), src_off, dst_prop,
                     zeros_128).reshape(NC, N, HID // 2)

    g2 = _tc2(scat1, g1, d8, b1.reshape(2, HID // 2), W2)
    scat2 = _prop_es(g2, src_es, dst_es,
                     zeros_128).reshape(NC, N, OUT_CH)

    return _tc3(scat2, g2, d8, b2.reshape(1, OUT_CH))


# speedup vs baseline: 1.0853x; 1.0853x over previous
import jax
import jax.numpy as jnp
from jax import lax
from jax.experimental import pallas as pl
from jax.experimental.pallas import tpu as pltpu
from jax.experimental.pallas import tpu_sc as plsc

N = 10000
E = 320000
IN_CH = 256
HID = 256
OUT_CH = 128

NC = 2
NS = 16
K = 80
NROWS = E // K
NBUF = 4
NBUF_PROP = 4
CR_PROP = 25
NCH_PROP = NROWS // NS // CR_PROP
CR_DEG = 25
NCH_DEG = NROWS // (NC * NS) // CR_DEG
CP_TILES = 10
CP_ROWS = N // CP_TILES
DEG_W = 128

_mesh = plsc.VectorSubcoreMesh(
    core_axis_name="c", subcore_axis_name="s", num_cores=NC, num_subcores=NS
)


def _zero_acc(sid, zeros_hbm, acc_sh):
    @pl.when(sid < CP_TILES)
    def _():
        pltpu.sync_copy(zeros_hbm,
                        acc_sh.at[pl.ds(sid * CP_ROWS, CP_ROWS)])


def _copy_out(cid, sid, out_hbm, acc_sh):
    @pl.when(sid < CP_TILES)
    def _():
        pltpu.sync_copy(acc_sh.at[pl.ds(sid * CP_ROWS, CP_ROWS)],
                        out_hbm.at[pl.ds(cid * N + sid * CP_ROWS, CP_ROWS)])


EPT = E // (NC * NS)


def _deg_body(dst_hbm, deg_out, dstv, acc):
    cid = lax.axis_index("c")
    sid = lax.axis_index("s")
    tid = cid * NS + sid

    zeros16 = jnp.zeros((16,), jnp.float32)

    def z(i, c):
        acc[pl.ds(i * 16, 16)] = zeros16
        return c

    lax.fori_loop(0, N // 16, z, 0)
    pltpu.sync_copy(dst_hbm.at[pl.ds(tid * EPT, EPT)], dstv)

    ones16 = jnp.ones((16,), jnp.float32)

    def step(i, c):
        idx = dstv[pl.ds(i * 16, 16)]
        plsc.addupdate_scatter(acc, [idx], ones16)
        return c

    lax.fori_loop(0, EPT // 16, step, 0)
    pltpu.sync_copy(acc, deg_out.at[pl.ds(tid * N, N)])


_deg_call = pl.kernel(
    _deg_body,
    out_type=jax.ShapeDtypeStruct((NC * NS * N,), jnp.float32),
    mesh=_mesh,
    scratch_types=[
        pltpu.VMEM((EPT,), jnp.int32),
        pltpu.VMEM((N,), jnp.float32),
    ],
    compiler_params=pltpu.CompilerParams(needs_layout_passes=False),
)


def _run_waves(cr, g_hbm, acc_sh, src_v, dst_v, bufs, gsems, ssems):
    nb = len(bufs)
    nwave = cr // nb
    ntail = cr - nwave * nb

    def wave(q, c2):
        base = q * nb
        for b in range(nb):
            j = base + b
            pltpu.make_async_copy(g_hbm.at[src_v.at[j]], bufs[b],
                                  gsems[b]).wait()
            pltpu.async_copy(bufs[b], acc_sh.at[dst_v.at[j]], ssems[b],
                             add=True)
        for b in range(nb):
            jn = base + b + nb

            @pl.when(jn < cr)
            def _(b=b, jn=jn):
                pltpu.make_async_copy(bufs[b], acc_sh.at[dst_v.at[0]],
                                      ssems[b]).wait()
                pltpu.async_copy(g_hbm.at[src_v.at[jn]], bufs[b],
                                 gsems[b])

        return c2

    lax.fori_loop(0, nwave, wave, 0)
    for b in range(ntail):
        j = nwave * nb + b
        pltpu.make_async_copy(g_hbm.at[src_v.at[j]], bufs[b],
                              gsems[b]).wait()
        pltpu.async_copy(bufs[b], acc_sh.at[dst_v.at[j]], ssems[b],
                         add=True)


def _drain_scatters(acc_sh, dst_v, bufs, ssems):
    for b in range(len(bufs)):
        pltpu.make_async_copy(bufs[b], acc_sh.at[dst_v.at[0]],
                              ssems[b]).wait()


def _prop_pipelined(nch, cr, src_row, dst_row, g_hbm, src_hbm, dst_hbm,
                    acc_sh, src_v, dst_v, bufs, gsems, ssems, is0, is1):
    nb = len(bufs)
    pltpu.sync_copy(src_hbm.at[src_row(0)], src_v)
    pltpu.sync_copy(dst_hbm.at[dst_row(0)], dst_v)
    for b in range(nb):
        pltpu.async_copy(g_hbm.at[src_v.at[b]], bufs[b], gsems[b])
    plsc.subcore_barrier()
    _run_waves(cr, g_hbm, acc_sh, src_v, dst_v, bufs, gsems, ssems)

    def chunk(t, carry):
        pltpu.async_copy(src_hbm.at[src_row(t)], src_v, is0)
        _drain_scatters(acc_sh, dst_v, bufs, ssems)
        pltpu.async_copy(dst_hbm.at[dst_row(t)], dst_v, is1)
        pltpu.make_async_copy(src_hbm.at[src_row(t)], src_v, is0).wait()
        for b in range(nb):
            pltpu.async_copy(g_hbm.at[src_v.at[b]], bufs[b], gsems[b])
        pltpu.make_async_copy(dst_hbm.at[dst_row(t)], dst_v, is1).wait()
        _run_waves(cr, g_hbm, acc_sh, src_v, dst_v, bufs, gsems, ssems)
        return carry

    lax.fori_loop(1, nch, chunk, 0)
    _drain_scatters(acc_sh, dst_v, bufs, ssems)


def _make_prop_body(nbuf, edge_split):
    def body(g_hbm, src_hbm, dst_hbm, zeros_hbm, out_hbm, *scr):
        src_v, dst_v = scr[0], scr[1]
        bufs = list(scr[2:2 + nbuf])
        acc_sh = scr[2 + nbuf]
        is0, is1 = scr[3 + nbuf], scr[4 + nbuf]
        gsems = list(scr[5 + nbuf:5 + 2 * nbuf])
        ssems = list(scr[5 + 2 * nbuf:5 + 3 * nbuf])

        cid = lax.axis_index("c")
        sid = lax.axis_index("s")
        tid = cid * NS + sid

        _zero_acc(sid, zeros_hbm, acc_sh)

        if edge_split:
            nch, cr = NCH_DEG, CR_DEG

            def src_row(t):
                return tid * nch + t

            dst_row = src_row
        else:
            nch, cr = NCH_PROP, CR_PROP

            def src_row(t):
                return tid * nch + t

            def dst_row(t):
                return sid * nch + t

        _prop_pipelined(nch, cr, src_row, dst_row, g_hbm, src_hbm, dst_hbm,
                        acc_sh, src_v, dst_v, bufs, gsems, ssems, is0, is1)
        plsc.subcore_barrier()
        _copy_out(cid, sid, out_hbm, acc_sh)

    return body


def _prop_scratch(c_width, cr, nbuf):
    return (
        [pltpu.VMEM((cr, K), jnp.int32),
         pltpu.VMEM((cr, K), jnp.int32)]
        + [pltpu.VMEM((K, c_width), jnp.float32) for _ in range(nbuf)]
        + [pltpu.VMEM_SHARED((N, c_width), jnp.float32)]
        + [pltpu.SemaphoreType.DMA for _ in range(2 + 2 * nbuf)]
    )


_prop128 = pl.kernel(
    _make_prop_body(NBUF_PROP, edge_split=False),
    out_type=jax.ShapeDtypeStruct((NC * N, HID // 2), jnp.float32),
    mesh=_mesh,
    scratch_types=_prop_scratch(HID // 2, CR_PROP, NBUF_PROP),
)


_prop_es = pl.kernel(
    _make_prop_body(NBUF, edge_split=True),
    out_type=jax.ShapeDtypeStruct((NC * N, OUT_CH), jnp.float32),
    mesh=_mesh,
    scratch_types=_prop_scratch(OUT_CH, CR_DEG, NBUF),
)


_B = 2000
_NB = N // _B
_PREC = lax.Precision.DEFAULT


def _tc1_body(dp_ref, x_ref, w_ref, g_ref, d8_ref):
    deg = jnp.sum(dp_ref[0], axis=0)[:, None] + 1.0
    d = lax.rsqrt(deg)
    h = jnp.dot(x_ref[...], w_ref[...], preferred_element_type=jnp.float32,
                precision=_PREC)
    g_ref[0] = h[:, :HID // 2] * d
    g_ref[1] = h[:, HID // 2:] * d
    d8_ref[...] = jnp.broadcast_to(d, (_B, 8))


def _tc1(degp, x, w1):
    return pl.pallas_call(
        _tc1_body,
        grid=(_NB,),
        in_specs=[
            pl.BlockSpec((1, NC * NS, _B), lambda i: (i, 0, 0)),
            pl.BlockSpec((_B, IN_CH), lambda i: (i, 0)),
            pl.BlockSpec((IN_CH, HID), lambda i: (0, 0)),
        ],
        out_specs=[
            pl.BlockSpec((2, _B, HID // 2), lambda i: (0, i, 0)),
            pl.BlockSpec((_B, 8), lambda i: (i, 0)),
        ],
        out_shape=[
            jax.ShapeDtypeStruct((2, N, HID // 2), jnp.float32),
            jax.ShapeDtypeStruct((N, 8), jnp.float32),
        ],
    )(degp, x, w1)


def _tc2_body(scat_ref, g_ref, d8_ref, b1_ref, w2_ref, out_ref):
    d = d8_ref[:, 0:1]
    h1a = jnp.maximum(d * (scat_ref[0] + g_ref[0]) + b1_ref[0:1, :], 0.0)
    h1b = jnp.maximum(d * (scat_ref[1] + g_ref[1]) + b1_ref[1:2, :], 0.0)
    res = (jnp.dot(h1a, w2_ref[0:HID // 2, :],
                   preferred_element_type=jnp.float32, precision=_PREC)
           + jnp.dot(h1b, w2_ref[HID // 2:, :],
                     preferred_element_type=jnp.float32, precision=_PREC))
    out_ref[...] = res * d


def _tc2(scat1, g1, d8, b1r, w2):
    return pl.pallas_call(
        _tc2_body,
        grid=(_NB,),
        in_specs=[
            pl.BlockSpec((2, _B, HID // 2), lambda i: (0, i, 0)),
            pl.BlockSpec((2, _B, HID // 2), lambda i: (0, i, 0)),
            pl.BlockSpec((_B, 8), lambda i: (i, 0)),
            pl.BlockSpec((2, HID // 2), lambda i: (0, 0)),
            pl.BlockSpec((HID, OUT_CH), lambda i: (0, 0)),
        ],
        out_specs=pl.BlockSpec((_B, OUT_CH), lambda i: (i, 0)),
        out_shape=jax.ShapeDtypeStruct((N, OUT_CH), jnp.float32),
    )(scat1, g1, d8, b1r, w2)


def _tc3_body(scat_ref, g_ref, d8_ref, b2_ref, z_ref):
    d = d8_ref[:, 0:1]
    z_ref[...] = d * (scat_ref[0] + scat_ref[1] + g_ref[...]) + b2_ref[0:1, :]


def _tc3(scat2, g2, d8, b2r):
    return pl.pallas_call(
        _tc3_body,
        grid=(_NB,),
        in_specs=[
            pl.BlockSpec((2, _B, OUT_CH), lambda i: (0, i, 0)),
            pl.BlockSpec((_B, OUT_CH), lambda i: (i, 0)),
            pl.BlockSpec((_B, 8), lambda i: (i, 0)),
            pl.BlockSpec((1, OUT_CH), lambda i: (0, 0)),
        ],
        out_specs=pl.BlockSpec((_B, OUT_CH), lambda i: (i, 0)),
        out_shape=jax.ShapeDtypeStruct((N, OUT_CH), jnp.float32),
    )(scat2, g2, d8, b2r)


def kernel(x, edge_index, W1, b1, W2, b2):
    src = edge_index[0].astype(jnp.int32)
    dst = edge_index[1].astype(jnp.int32)
    src_off = jnp.concatenate([src, src + N]).reshape(
        NC * NS * NCH_PROP, CR_PROP, K)
    dst_prop = dst.reshape(NS * NCH_PROP, CR_PROP, K)
    src_es = src.reshape(NC * NS * NCH_DEG, CR_DEG, K)
    dst_es = dst.reshape(NC * NS * NCH_DEG, CR_DEG, K)

    zeros_128 = jnp.zeros((CP_ROWS, HID // 2), jnp.float32)

    degp = _deg_call(dst).reshape(NC * NS, _NB, _B).transpose(1, 0, 2)

    g1, d8 = _tc1(degp, x, W1)
    scat1 = _prop128(g1.reshape(NC * N, HID // 2), src_off, dst_prop,
                     zeros_128).reshape(NC, N, HID // 2)

    g2 = _tc2(scat1, g1, d8, b1.reshape(2, HID // 2), W2)
    scat2 = _prop_es(g2, src_es, dst_es,
                     zeros_128).reshape(NC, N, OUT_CH)

    return _tc3(scat2, g2, d8, b2.reshape(1, OUT_CH))


# speedup vs baseline: 1.0909x; 1.0051x over previous
import jax
import jax.numpy as jnp
from jax import lax
from jax.experimental import pallas as pl
from jax.experimental.pallas import tpu as pltpu
from jax.experimental.pallas import tpu_sc as plsc

N = 10000
E = 320000
IN_CH = 256
HID = 256
OUT_CH = 128

NC = 2
NS = 16
K = 80
NROWS = E // K
NBUF = 4
NBUF_PROP = 4
CR_PROP = 25
NCH_PROP = NROWS // NS // CR_PROP
CR_DEG = 25
NCH_DEG = NROWS // (NC * NS) // CR_DEG
CP_TILES = 10
CP_ROWS = N // CP_TILES
DEG_W = 128

_mesh = plsc.VectorSubcoreMesh(
    core_axis_name="c", subcore_axis_name="s", num_cores=NC, num_subcores=NS
)


def _zero_acc(sid, zeros_hbm, acc_sh):
    @pl.when(sid < CP_TILES)
    def _():
        pltpu.sync_copy(zeros_hbm,
                        acc_sh.at[pl.ds(sid * CP_ROWS, CP_ROWS)])


def _copy_out(cid, sid, out_hbm, acc_sh):
    @pl.when(sid < CP_TILES)
    def _():
        pltpu.sync_copy(acc_sh.at[pl.ds(sid * CP_ROWS, CP_ROWS)],
                        out_hbm.at[pl.ds(cid * N + sid * CP_ROWS, CP_ROWS)])


EPT = E // (NC * NS)


def _deg_body(dst_hbm, deg_out, dstv, acc):
    cid = lax.axis_index("c")
    sid = lax.axis_index("s")
    tid = cid * NS + sid

    zeros16 = jnp.zeros((16,), jnp.float32)

    def z(i, c):
        acc[pl.ds(i * 16, 16)] = zeros16
        return c

    lax.fori_loop(0, N // 16, z, 0)
    pltpu.sync_copy(dst_hbm.at[pl.ds(tid * EPT, EPT)], dstv)

    ones16 = jnp.ones((16,), jnp.float32)

    def step(i, c):
        idx = dstv[pl.ds(i * 16, 16)]
        plsc.addupdate_scatter(acc, [idx], ones16)
        return c

    lax.fori_loop(0, EPT // 16, step, 0)
    pltpu.sync_copy(acc, deg_out.at[pl.ds(tid * N, N)])


_deg_call = pl.kernel(
    _deg_body,
    out_type=jax.ShapeDtypeStruct((NC * NS * N,), jnp.float32),
    mesh=_mesh,
    scratch_types=[
        pltpu.VMEM((EPT,), jnp.int32),
        pltpu.VMEM((N,), jnp.float32),
    ],
    compiler_params=pltpu.CompilerParams(needs_layout_passes=False),
)


def _run_waves(cr, g_hbm, acc_sh, src_v, dst_v, bufs, gsems, ssems):
    nb = len(bufs)
    nwave = cr // nb
    ntail = cr - nwave * nb

    def wave(q, c2):
        base = q * nb
        for b in range(nb):
            j = base + b
            pltpu.make_async_copy(g_hbm.at[src_v.at[j]], bufs[b],
                                  gsems[b]).wait()
            pltpu.async_copy(bufs[b], acc_sh.at[dst_v.at[j]], ssems[b],
                             add=True)
        for b in range(nb):
            jn = base + b + nb

            @pl.when(jn < cr)
            def _(b=b, jn=jn):
                pltpu.make_async_copy(bufs[b], acc_sh.at[dst_v.at[0]],
                                      ssems[b]).wait()
                pltpu.async_copy(g_hbm.at[src_v.at[jn]], bufs[b],
                                 gsems[b])

        return c2

    lax.fori_loop(0, nwave, wave, 0)
    for b in range(ntail):
        j = nwave * nb + b
        pltpu.make_async_copy(g_hbm.at[src_v.at[j]], bufs[b],
                              gsems[b]).wait()
        pltpu.async_copy(bufs[b], acc_sh.at[dst_v.at[j]], ssems[b],
                         add=True)


def _drain_scatters(acc_sh, dst_v, bufs, ssems):
    for b in range(len(bufs)):
        pltpu.make_async_copy(bufs[b], acc_sh.at[dst_v.at[0]],
                              ssems[b]).wait()


def _prop_pipelined(nch, cr, src_row, dst_row, g_hbm, src_hbm, dst_hbm,
                    acc_sh, src_v, dst_v, bufs, gsems, ssems, is0, is1):
    nb = len(bufs)
    pltpu.sync_copy(src_hbm.at[src_row(0)], src_v)
    pltpu.sync_copy(dst_hbm.at[dst_row(0)], dst_v)
    for b in range(nb):
        pltpu.async_copy(g_hbm.at[src_v.at[b]], bufs[b], gsems[b])
    plsc.subcore_barrier()
    _run_waves(cr, g_hbm, acc_sh, src_v, dst_v, bufs, gsems, ssems)

    def chunk(t, carry):
        pltpu.async_copy(src_hbm.at[src_row(t)], src_v, is0)
        _drain_scatters(acc_sh, dst_v, bufs, ssems)
        pltpu.async_copy(dst_hbm.at[dst_row(t)], dst_v, is1)
        pltpu.make_async_copy(src_hbm.at[src_row(t)], src_v, is0).wait()
        for b in range(nb):
            pltpu.async_copy(g_hbm.at[src_v.at[b]], bufs[b], gsems[b])
        pltpu.make_async_copy(dst_hbm.at[dst_row(t)], dst_v, is1).wait()
        _run_waves(cr, g_hbm, acc_sh, src_v, dst_v, bufs, gsems, ssems)
        return carry

    lax.fori_loop(1, nch, chunk, 0)
    _drain_scatters(acc_sh, dst_v, bufs, ssems)


def _make_prop_body(nbuf, edge_split):
    def body(g_hbm, src_hbm, dst_hbm, zeros_hbm, out_hbm, *scr):
        src_v, dst_v = scr[0], scr[1]
        bufs = list(scr[2:2 + nbuf])
        acc_sh = scr[2 + nbuf]
        is0, is1 = scr[3 + nbuf], scr[4 + nbuf]
        gsems = list(scr[5 + nbuf:5 + 2 * nbuf])
        ssems = list(scr[5 + 2 * nbuf:5 + 3 * nbuf])

        cid = lax.axis_index("c")
        sid = lax.axis_index("s")
        tid = cid * NS + sid

        _zero_acc(sid, zeros_hbm, acc_sh)

        if edge_split:
            nch, cr = NCH_DEG, CR_DEG

            def src_row(t):
                return tid * nch + t

            dst_row = src_row
        else:
            nch, cr = NCH_PROP, CR_PROP

            def src_row(t):
                return tid * nch + t

            def dst_row(t):
                return sid * nch + t

        _prop_pipelined(nch, cr, src_row, dst_row, g_hbm, src_hbm, dst_hbm,
                        acc_sh, src_v, dst_v, bufs, gsems, ssems, is0, is1)
        plsc.subcore_barrier()
        _copy_out(cid, sid, out_hbm, acc_sh)

    return body


def _prop_scratch(c_width, cr, nbuf):
    return (
        [pltpu.VMEM((cr, K), jnp.int32),
         pltpu.VMEM((cr, K), jnp.int32)]
        + [pltpu.VMEM((K, c_width), jnp.float32) for _ in range(nbuf)]
        + [pltpu.VMEM_SHARED((N, c_width), jnp.float32)]
        + [pltpu.SemaphoreType.DMA for _ in range(2 + 2 * nbuf)]
    )


_prop128 = pl.kernel(
    _make_prop_body(NBUF_PROP, edge_split=False),
    out_type=jax.ShapeDtypeStruct((NC * N, HID // 2), jnp.float32),
    mesh=_mesh,
    scratch_types=_prop_scratch(HID // 2, CR_PROP, NBUF_PROP),
)


_prop_es = pl.kernel(
    _make_prop_body(NBUF, edge_split=True),
    out_type=jax.ShapeDtypeStruct((NC * N, OUT_CH), jnp.float32),
    mesh=_mesh,
    scratch_types=_prop_scratch(OUT_CH, CR_DEG, NBUF),
)


_B = 5000
_NB = N // _B
_PREC = lax.Precision.DEFAULT


def _tc1_body(dp_ref, x_ref, w_ref, g_ref, d8_ref):
    deg = jnp.sum(dp_ref[0], axis=0)[:, None] + 1.0
    d = lax.rsqrt(deg)
    h = jnp.dot(x_ref[...], w_ref[...], preferred_element_type=jnp.float32,
                precision=_PREC)
    g_ref[0] = h[:, :HID // 2] * d
    g_ref[1] = h[:, HID // 2:] * d
    d8_ref[...] = jnp.broadcast_to(d, (_B, 8))


def _tc1(degp, x, w1):
    return pl.pallas_call(
        _tc1_body,
        grid=(_NB,),
        in_specs=[
            pl.BlockSpec((1, NC * NS, _B), lambda i: (i, 0, 0)),
            pl.BlockSpec((_B, IN_CH), lambda i: (i, 0)),
            pl.BlockSpec((IN_CH, HID), lambda i: (0, 0)),
        ],
        out_specs=[
            pl.BlockSpec((2, _B, HID // 2), lambda i: (0, i, 0)),
            pl.BlockSpec((_B, 8), lambda i: (i, 0)),
        ],
        out_shape=[
            jax.ShapeDtypeStruct((2, N, HID // 2), jnp.float32),
            jax.ShapeDtypeStruct((N, 8), jnp.float32),
        ],
    )(degp, x, w1)


def _tc2_body(scat_ref, g_ref, d8_ref, b1_ref, w2_ref, out_ref):
    d = d8_ref[:, 0:1]
    h1a = jnp.maximum(d * (scat_ref[0] + g_ref[0]) + b1_ref[0:1, :], 0.0)
    h1b = jnp.maximum(d * (scat_ref[1] + g_ref[1]) + b1_ref[1:2, :], 0.0)
    res = (jnp.dot(h1a, w2_ref[0:HID // 2, :],
                   preferred_element_type=jnp.float32, precision=_PREC)
           + jnp.dot(h1b, w2_ref[HID // 2:, :],
                     preferred_element_type=jnp.float32, precision=_PREC))
    out_ref[...] = res * d


def _tc2(scat1, g1, d8, b1r, w2):
    return pl.pallas_call(
        _tc2_body,
        grid=(_NB,),
        in_specs=[
            pl.BlockSpec((2, _B, HID // 2), lambda i: (0, i, 0)),
            pl.BlockSpec((2, _B, HID // 2), lambda i: (0, i, 0)),
            pl.BlockSpec((_B, 8), lambda i: (i, 0)),
            pl.BlockSpec((2, HID // 2), lambda i: (0, 0)),
            pl.BlockSpec((HID, OUT_CH), lambda i: (0, 0)),
        ],
        out_specs=pl.BlockSpec((_B, OUT_CH), lambda i: (i, 0)),
        out_shape=jax.ShapeDtypeStruct((N, OUT_CH), jnp.float32),
    )(scat1, g1, d8, b1r, w2)


def _tc3_body(scat_ref, g_ref, d8_ref, b2_ref, z_ref):
    d = d8_ref[:, 0:1]
    z_ref[...] = d * (scat_ref[0] + scat_ref[1] + g_ref[...]) + b2_ref[0:1, :]


def _tc3(scat2, g2, d8, b2r):
    return pl.pallas_call(
        _tc3_body,
        grid=(_NB,),
        in_specs=[
            pl.BlockSpec((2, _B, OUT_CH), lambda i: (0, i, 0)),
            pl.BlockSpec((_B, OUT_CH), lambda i: (i, 0)),
            pl.BlockSpec((_B, 8), lambda i: (i, 0)),
            pl.BlockSpec((1, OUT_CH), lambda i: (0, 0)),
        ],
        out_specs=pl.BlockSpec((_B, OUT_CH), lambda i: (i, 0)),
        out_shape=jax.ShapeDtypeStruct((N, OUT_CH), jnp.float32),
    )(scat2, g2, d8, b2r)


def kernel(x, edge_index, W1, b1, W2, b2):
    src = edge_index[0].astype(jnp.int32)
    dst = edge_index[1].astype(jnp.int32)
    src_off = jnp.concatenate([src, src + N]).reshape(
        NC * NS * NCH_PROP, CR_PROP, K)
    dst_prop = dst.reshape(NS * NCH_PROP, CR_PROP, K)
    src_es = src.reshape(NC * NS * NCH_DEG, CR_DEG, K)
    dst_es = dst.reshape(NC * NS * NCH_DEG, CR_DEG, K)

    zeros_128 = jnp.zeros((CP_ROWS, HID // 2), jnp.float32)

    degp = _deg_call(dst).reshape(NC * NS, _NB, _B).transpose(1, 0, 2)

    g1, d8 = _tc1(degp, x, W1)
    scat1 = _prop128(g1.reshape(NC * N, HID // 2), src_off, dst_prop,
                     zeros_128).reshape(NC, N, HID // 2)

    g2 = _tc2(scat1, g1, d8, b1.reshape(2, HID // 2), W2)
    scat2 = _prop_es(g2, src_es, dst_es,
                     zeros_128).reshape(NC, N, OUT_CH)

    return _tc3(scat2, g2, d8, b2.reshape(1, OUT_CH))
